# Initial kernel scaffold; baseline (speedup 1.0000x reference)
import jax
import jax.numpy as jnp
from jax.experimental import pallas as pl


def kernel(features, edge_index, edge_attr, W1, b1, W2, b2, Wd, bd, Wf, bf):
    raise NotImplementedError("write your pallas kernel here")


# speedup vs baseline: 150.5418x; 150.5418x over previous
import functools

import jax
import jax.numpy as jnp
from jax import lax
from jax.experimental import pallas as pl
from jax.experimental.pallas import tpu as pltpu
from jax.experimental.pallas import tpu_sc as plsc

N = 4096
E = 131072
NC, NS = 2, 16
NW = NC * NS
EPW = E // NW
CHUNK = 128
NCHUNK = EPW // CHUNK
RPT = N // NS

f32 = jnp.float32
i32 = jnp.int32


def _wid():
    cid = jnp.int32(lax.axis_index("c"))
    sid = jnp.int32(lax.axis_index("s"))
    return cid, sid, sid * jnp.int32(NC) + cid


def _sc_mesh():
    return plsc.VectorSubcoreMesh(
        core_axis_name="c", subcore_axis_name="s",
        num_cores=NC, num_subcores=NS)


_SC_PARAMS = pltpu.CompilerParams(use_tc_tiling_on_sc=False)


def _deg_body(col_hbm, ew_hbm, zero_hbm, out_hbm, colv, ewv, buf, acc):
    cid, sid, wid = _wid()
    pltpu.sync_copy(zero_hbm.at[pl.ds(sid * i32(RPT), RPT)],
                    acc.at[pl.ds(sid * i32(RPT), RPT)])
    plsc.subcore_barrier()

    def chunk(i, carry):
        base = pl.multiple_of(wid * i32(EPW) + i * i32(CHUNK), CHUNK)
        pltpu.sync_copy(col_hbm.at[pl.ds(base, CHUNK)], colv)
        pltpu.sync_copy(ew_hbm.at[pl.ds(base, CHUNK)], ewv)

        def fill(g, c2):
            ews = ewv[pl.ds(g * i32(16), 16)]
            for r in range(16):
                buf[g * i32(16) + i32(r), :] = jnp.full((16,), ews[r],
                                                        dtype=f32)
            return c2
        lax.fori_loop(i32(0), i32(CHUNK // 16), fill, i32(0))
        pltpu.sync_copy(buf, acc.at[colv], add=True)
        return carry
    lax.fori_loop(i32(0), i32(NCHUNK), chunk, i32(0))
    plsc.subcore_barrier()
    pltpu.sync_copy(acc.at[pl.ds(sid * i32(RPT), RPT)],
                    out_hbm.at[cid, pl.ds(sid * i32(RPT), RPT)])


def _deg_call(col, ew, zero16):
    return pl.kernel(
        _deg_body,
        out_type=jax.ShapeDtypeStruct((NC, N, 16), f32),
        mesh=_sc_mesh(),
        scratch_types=[
            pltpu.VMEM((CHUNK,), jnp.int32),
            pltpu.VMEM((CHUNK,), f32),
            pltpu.VMEM((CHUNK, 16), f32),
            pltpu.VMEM_SHARED((N, 16), f32),
        ],
        compiler_params=_SC_PARAMS,
    )(col, ew, zero16)


def _conv_body(D, hs_hbm, row_hbm, col_hbm, ew_hbm, zero_hbm, out_hbm,
               rowv, colv, ewv, buf, acc):
    cid, sid, wid = _wid()
    pltpu.sync_copy(zero_hbm.at[pl.ds(sid * i32(RPT), RPT)],
                    acc.at[pl.ds(sid * i32(RPT), RPT)])
    plsc.subcore_barrier()
    nvpr = D // 16

    def chunk(i, carry):
        base = pl.multiple_of(wid * i32(EPW) + i * i32(CHUNK), CHUNK)
        pltpu.sync_copy(row_hbm.at[pl.ds(base, CHUNK)], rowv)
        pltpu.sync_copy(col_hbm.at[pl.ds(base, CHUNK)], colv)
        pltpu.sync_copy(ew_hbm.at[pl.ds(base, CHUNK)], ewv)
        pltpu.sync_copy(hs_hbm.at[rowv], buf)

        def scale(g, c2):
            ews = ewv[pl.ds(g * i32(16), 16)]
            for r in range(16):
                rr = g * i32(16) + i32(r)
                s = ews[r]
                for j in range(nvpr):
                    buf[rr, pl.ds(j * 16, 16)] = buf[rr, pl.ds(j * 16, 16)] * s
            return c2
        lax.fori_loop(i32(0), i32(CHUNK // 16), scale, i32(0))
        pltpu.sync_copy(buf, acc.at[colv], add=True)
        return carry
    lax.fori_loop(i32(0), i32(NCHUNK), chunk, i32(0))
    plsc.subcore_barrier()
    pltpu.sync_copy(acc.at[pl.ds(sid * i32(RPT), RPT)],
                    out_hbm.at[cid, pl.ds(sid * i32(RPT), RPT)])


def _conv_call(D, hs, row, col, ew, zeroD):
    return pl.kernel(
        functools.partial(_conv_body, D),
        out_type=jax.ShapeDtypeStruct((NC, N, D), f32),
        mesh=_sc_mesh(),
        scratch_types=[
            pltpu.VMEM((CHUNK,), jnp.int32),
            pltpu.VMEM((CHUNK,), jnp.int32),
            pltpu.VMEM((CHUNK,), f32),
            pltpu.VMEM((CHUNK, D), f32),
            pltpu.VMEM_SHARED((N, D), f32),
        ],
        compiler_params=_SC_PARAMS,
    )(hs, row, col, ew, zeroD)


def _dis(degp_ref):
    deg = (degp_ref[0] + degp_ref[1])[:, 0:1] + 1.0
    return lax.rsqrt(deg)


def _tc1_body(x_ref, w1_ref, degp_ref, hs1_ref):
    hs1_ref[...] = _dis(degp_ref) * jnp.dot(
        x_ref[...], w1_ref[...], preferred_element_type=f32)


def _tc2_body(degp_ref, acc1_ref, hs1_ref, w2_ref, b1_ref, hs2_ref):
    dis = _dis(degp_ref)
    h1 = jax.nn.relu(dis * (acc1_ref[0] + acc1_ref[1] + hs1_ref[...])
                     + b1_ref[...])
    hs2_ref[...] = dis * jnp.dot(h1, w2_ref[...], preferred_element_type=f32)


def _tc3_body(degp_ref, acc2_ref, hs2_ref, b2_ref, x_ref, wd_ref, hs3_ref):
    dis = _dis(degp_ref)
    z = jnp.tanh(dis * (acc2_ref[0] + acc2_ref[1] + hs2_ref[...])
                 + b2_ref[...])
    enc = jnp.concatenate([z, z, z, z], axis=1) + x_ref[...]
    hs3_ref[...] = dis * jnp.dot(enc, wd_ref[...], preferred_element_type=f32)


def _tc4_body(degp_ref, acc3_ref, hs3_ref, bd_ref, wf_ref, bf_ref, out_ref):
    dis = _dis(degp_ref)
    d = jax.nn.relu(dis * (acc3_ref[0] + acc3_ref[1] + hs3_ref[...])
                    + bd_ref[...])
    s = jnp.sum(d * wf_ref[...], keepdims=True).reshape(1, 1) + bf_ref[...]
    out_ref[...] = jax.nn.sigmoid(s)


def _tc_call(body, out_shape, *args):
    return pl.pallas_call(
        body, out_shape=jax.ShapeDtypeStruct(out_shape, f32))(*args)


def kernel(features, edge_index, edge_attr, W1, b1, W2, b2, Wd, bd, Wf, bf):
    x = features.astype(f32)
    ew = edge_attr.astype(f32)
    row = edge_index[0].astype(jnp.int32)
    col = edge_index[1].astype(jnp.int32)
    w1 = W1.astype(f32)
    b1f = b1.astype(f32).reshape(1, -1)
    w2 = W2.astype(f32)
    b2f = b2.astype(f32).reshape(1, -1)
    wd = jnp.zeros((x.shape[1], 16), f32).at[:, :2].set(Wd.astype(f32))
    bdp = jnp.zeros((1, 16), f32).at[0, :2].set(bd.astype(f32))
    wf = jnp.zeros((N, 16), f32).at[:, :2].set(Wf.astype(f32).reshape(N, 2))
    bff = bf.astype(f32).reshape(1, 1)

    zero16 = jnp.zeros((N, 16), f32)
    zero64 = jnp.zeros((N, 64), f32)
    zero32 = jnp.zeros((N, 32), f32)

    degp = _deg_call(col, ew, zero16)
    hs1 = _tc_call(_tc1_body, (N, 64), x, w1, degp)
    acc1 = _conv_call(64, hs1, row, col, ew, zero64)
    hs2 = _tc_call(_tc2_body, (N, 32), degp, acc1, hs1, w2, b1f)
    acc2 = _conv_call(32, hs2, row, col, ew, zero32)
    hs3 = _tc_call(_tc3_body, (N, 16), degp, acc2, hs2, b2f, x, wd)
    acc3 = _conv_call(16, hs3, row, col, ew, zero16)
    out = _tc_call(_tc4_body, (1, 1), degp, acc3, hs3, bdp, wf, bff)
    return out.reshape(()).astype(jnp.float64)


# speedup vs baseline: 269.3578x; 1.7893x over previous
import functools

import jax
import jax.numpy as jnp
from jax import lax
from jax.experimental import pallas as pl
from jax.experimental.pallas import tpu as pltpu
from jax.experimental.pallas import tpu_sc as plsc

N = 4096
E = 131072
NC, NS = 2, 16
NW = NC * NS
EPW = E // NW
CHUNK = 128
NCHUNK = EPW // CHUNK
RPT = N // NS

f32 = jnp.float32
i32 = jnp.int32

_SC_PARAMS = pltpu.CompilerParams(use_tc_tiling_on_sc=False)


def _wid():
    cid = jnp.int32(lax.axis_index("c"))
    sid = jnp.int32(lax.axis_index("s"))
    return cid, sid, sid * jnp.int32(NC) + cid


def _sc_mesh():
    return plsc.VectorSubcoreMesh(
        core_axis_name="c", subcore_axis_name="s",
        num_cores=NC, num_subcores=NS)


def _deg_body(col_hbm, ew_hbm, zero_hbm, out_hbm, colv, ewv, buf, s0, s1,
              acc):
    cid, sid, wid = _wid()
    pltpu.sync_copy(zero_hbm.at[pl.ds(sid * i32(RPT), RPT)],
                    acc.at[pl.ds(sid * i32(RPT), RPT)])
    pltpu.sync_copy(col_hbm.at[wid], colv)
    pltpu.sync_copy(ew_hbm.at[wid], ewv)
    plsc.subcore_barrier()
    sems = (s0, s1)

    def scatter(i, b):
        pltpu.async_copy(buf.at[i32(b)], acc.at[colv.at[i]], sems[b], add=True)

    def wait_scatter(i, b):
        pltpu.make_async_copy(buf.at[i32(b)], acc.at[colv.at[i]], sems[b]).wait()

    def fill(i, b):
        def grp(g, c2):
            ews = ewv[i, pl.ds(g * i32(16), 16)]
            for r in range(16):
                buf[b, g * i32(16) + i32(r), :] = jnp.full((16,), ews[r],
                                                           dtype=f32)
            return c2
        lax.fori_loop(i32(0), i32(CHUNK // 16), grp, i32(0))

    def outer(io, carry):
        for b in range(2):
            i = io * i32(2) + i32(b)

            @pl.when(io > i32(0))
            def _():
                wait_scatter(i - i32(2), b)
            fill(i, b)
            scatter(i, b)
        return carry
    lax.fori_loop(i32(0), i32(NCHUNK // 2), outer, i32(0))
    wait_scatter(i32(NCHUNK - 2), 0)
    wait_scatter(i32(NCHUNK - 1), 1)
    plsc.subcore_barrier()
    pltpu.sync_copy(acc.at[pl.ds(sid * i32(RPT), RPT)],
                    out_hbm.at[cid, pl.ds(sid * i32(RPT), RPT)])


def _deg_call(col3, ew3, zero16):
    return pl.kernel(
        _deg_body,
        out_type=jax.ShapeDtypeStruct((NC, N, 16), f32),
        mesh=_sc_mesh(),
        scratch_types=[
            pltpu.VMEM((NCHUNK, CHUNK), jnp.int32),
            pltpu.VMEM((NCHUNK, CHUNK), f32),
            pltpu.VMEM((2, CHUNK, 16), f32),
            pltpu.SemaphoreType.DMA,
            pltpu.SemaphoreType.DMA,
            pltpu.VMEM_SHARED((N, 16), f32),
        ],
        compiler_params=_SC_PARAMS,
    )(col3, ew3, zero16)


def _conv_body(D, hs_hbm, row_hbm, col_hbm, ew_hbm, zero_hbm, out_hbm,
               rowv, colv, ewv, buf, g0, g1, s0, s1, acc):
    cid, sid, wid = _wid()
    pltpu.sync_copy(zero_hbm.at[pl.ds(sid * i32(RPT), RPT)],
                    acc.at[pl.ds(sid * i32(RPT), RPT)])
    pltpu.sync_copy(row_hbm.at[wid], rowv)
    pltpu.sync_copy(col_hbm.at[wid], colv)
    pltpu.sync_copy(ew_hbm.at[wid], ewv)
    plsc.subcore_barrier()
    nvpr = D // 16
    gsems = (g0, g1)
    ssems = (s0, s1)

    def gather(i, b):
        pltpu.async_copy(hs_hbm.at[rowv.at[i]], buf.at[i32(b)], gsems[b])

    def wait_gather(i, b):
        pltpu.make_async_copy(hs_hbm.at[rowv.at[i]], buf.at[i32(b)],
                              gsems[b]).wait()

    def scatter(i, b):
        pltpu.async_copy(buf.at[i32(b)], acc.at[colv.at[i]], ssems[b], add=True)

    def wait_scatter(i, b):
        pltpu.make_async_copy(buf.at[i32(b)], acc.at[colv.at[i]],
                              ssems[b]).wait()

    def scale(i, b):
        def grp(g, c2):
            ews = ewv[i, pl.ds(g * i32(16), 16)]
            for r in range(16):
                rr = g * i32(16) + i32(r)
                s = ews[r]
                for j in range(nvpr):
                    buf[b, rr, pl.ds(j * 16, 16)] = (
                        buf[b, rr, pl.ds(j * 16, 16)] * s)
            return c2
        lax.fori_loop(i32(0), i32(CHUNK // 16), grp, i32(0))

    gather(i32(0), 0)

    def outer(io, carry):
        for b in range(2):
            i = io * i32(2) + i32(b)
            if b == 0:
                @pl.when(io > i32(0))
                def _():
                    wait_scatter(i - i32(1), 1)
                gather(i + i32(1), 1)
            else:
                wait_scatter(i - i32(1), 0)

                @pl.when(io < i32(NCHUNK // 2 - 1))
                def _():
                    gather(i + i32(1), 0)
            wait_gather(i, b)
            scale(i, b)
            scatter(i, b)
        return carry
    lax.fori_loop(i32(0), i32(NCHUNK // 2), outer, i32(0))
    wait_scatter(i32(NCHUNK - 1), 1)
    plsc.subcore_barrier()
    pltpu.sync_copy(acc.at[pl.ds(sid * i32(RPT), RPT)],
                    out_hbm.at[cid, pl.ds(sid * i32(RPT), RPT)])


def _conv_call(D, hs, row3, col3, ew3, zeroD):
    return pl.kernel(
        functools.partial(_conv_body, D),
        out_type=jax.ShapeDtypeStruct((NC, N, D), f32),
        mesh=_sc_mesh(),
        scratch_types=[
            pltpu.VMEM((NCHUNK, CHUNK), jnp.int32),
            pltpu.VMEM((NCHUNK, CHUNK), jnp.int32),
            pltpu.VMEM((NCHUNK, CHUNK), f32),
            pltpu.VMEM((2, CHUNK, D), f32),
            pltpu.SemaphoreType.DMA,
            pltpu.SemaphoreType.DMA,
            pltpu.SemaphoreType.DMA,
            pltpu.SemaphoreType.DMA,
            pltpu.VMEM_SHARED((N, D), f32),
        ],
        compiler_params=_SC_PARAMS,
    )(hs, row3, col3, ew3, zeroD)


def _dis(degp_ref):
    deg = (degp_ref[0] + degp_ref[1])[:, 0:1] + 1.0
    return lax.rsqrt(deg)


def _tc1_body(x_ref, w1_ref, degp_ref, hs1_ref):
    hs1_ref[...] = _dis(degp_ref) * jnp.dot(
        x_ref[...], w1_ref[...], preferred_element_type=f32)


def _tc2_body(degp_ref, acc1_ref, hs1_ref, w2_ref, b1_ref, hs2_ref):
    dis = _dis(degp_ref)
    h1 = jax.nn.relu(dis * (acc1_ref[0] + acc1_ref[1] + hs1_ref[...])
                     + b1_ref[...])
    hs2_ref[...] = dis * jnp.dot(h1, w2_ref[...], preferred_element_type=f32)


def _tc3_body(degp_ref, acc2_ref, hs2_ref, b2_ref, x_ref, wd_ref, hs3_ref):
    dis = _dis(degp_ref)
    z = jnp.tanh(dis * (acc2_ref[0] + acc2_ref[1] + hs2_ref[...])
                 + b2_ref[...])
    enc = jnp.concatenate([z, z, z, z], axis=1) + x_ref[...]
    hs3_ref[...] = dis * jnp.dot(enc, wd_ref[...], preferred_element_type=f32)


def _tc4_body(degp_ref, acc3_ref, hs3_ref, bd_ref, wf_ref, bf_ref, out_ref):
    dis = _dis(degp_ref)
    d = jax.nn.relu(dis * (acc3_ref[0] + acc3_ref[1] + hs3_ref[...])
                    + bd_ref[...])
    s = jnp.sum(d * wf_ref[...], keepdims=True).reshape(1, 1) + bf_ref[...]
    out_ref[...] = jax.nn.sigmoid(s)


def _tc_call(body, out_shape, *args):
    return pl.pallas_call(
        body, out_shape=jax.ShapeDtypeStruct(out_shape, f32))(*args)


def kernel(features, edge_index, edge_attr, W1, b1, W2, b2, Wd, bd, Wf, bf):
    x = features.astype(f32)
    ew3 = edge_attr.astype(f32).reshape(NW, NCHUNK, CHUNK)
    row3 = edge_index[0].astype(jnp.int32).reshape(NW, NCHUNK, CHUNK)
    col3 = edge_index[1].astype(jnp.int32).reshape(NW, NCHUNK, CHUNK)
    w1 = W1.astype(f32)
    b1f = b1.astype(f32).reshape(1, -1)
    w2 = W2.astype(f32)
    b2f = b2.astype(f32).reshape(1, -1)
    wd = jnp.zeros((x.shape[1], 16), f32).at[:, :2].set(Wd.astype(f32))
    bdp = jnp.zeros((1, 16), f32).at[0, :2].set(bd.astype(f32))
    wf = jnp.zeros((N, 16), f32).at[:, :2].set(Wf.astype(f32).reshape(N, 2))
    bff = bf.astype(f32).reshape(1, 1)

    zero16 = jnp.zeros((N, 16), f32)
    zero64 = jnp.zeros((N, 64), f32)
    zero32 = jnp.zeros((N, 32), f32)

    degp = _deg_call(col3, ew3, zero16)
    hs1 = _tc_call(_tc1_body, (N, 64), x, w1, degp)
    acc1 = _conv_call(64, hs1, row3, col3, ew3, zero64)
    hs2 = _tc_call(_tc2_body, (N, 32), degp, acc1, hs1, w2, b1f)
    acc2 = _conv_call(32, hs2, row3, col3, ew3, zero32)
    hs3 = _tc_call(_tc3_body, (N, 16), degp, acc2, hs2, b2f, x, wd)
    acc3 = _conv_call(16, hs3, row3, col3, ew3, zero16)
    out = _tc_call(_tc4_body, (1, 1), degp, acc3, hs3, bdp, wf, bff)
    return out.reshape(()).astype(jnp.float64)


# speedup vs baseline: 341.1041x; 1.2664x over previous
import functools

import jax
import jax.numpy as jnp
from jax import lax
from jax.experimental import pallas as pl
from jax.experimental.pallas import tpu as pltpu
from jax.experimental.pallas import tpu_sc as plsc

N = 4096
E = 131072
NC, NS = 2, 16
NW = NC * NS
EPW = E // NW
CHUNK = 128
NCHUNK = EPW // CHUNK
RPT = N // NS

f32 = jnp.float32
i32 = jnp.int32

_SC_PARAMS = pltpu.CompilerParams(use_tc_tiling_on_sc=False)


def _wid():
    cid = jnp.int32(lax.axis_index("c"))
    sid = jnp.int32(lax.axis_index("s"))
    return cid, sid, sid * jnp.int32(NC) + cid


def _sc_mesh():
    return plsc.VectorSubcoreMesh(
        core_axis_name="c", subcore_axis_name="s",
        num_cores=NC, num_subcores=NS)


def _deg_body(col_hbm, ew_hbm, zero_hbm, out_hbm, colv, ewv, buf, s0, s1,
              acc):
    cid, sid, wid = _wid()
    pltpu.sync_copy(zero_hbm.at[pl.ds(sid * i32(RPT), RPT)],
                    acc.at[pl.ds(sid * i32(RPT), RPT)])
    pltpu.sync_copy(col_hbm.at[wid], colv)
    pltpu.sync_copy(ew_hbm.at[wid], ewv)
    plsc.subcore_barrier()
    sems = (s0, s1)

    def scatter(i, b):
        pltpu.async_copy(buf.at[i32(b)], acc.at[colv.at[i]], sems[b], add=True)

    def wait_scatter(i, b):
        pltpu.make_async_copy(buf.at[i32(b)], acc.at[colv.at[i]], sems[b]).wait()

    def fill(i, b):
        def grp(g, c2):
            ews = ewv[i, pl.ds(g * i32(16), 16)]
            for r in range(16):
                buf[b, g * i32(16) + i32(r), :] = jnp.full((16,), ews[r],
                                                           dtype=f32)
            return c2
        lax.fori_loop(i32(0), i32(CHUNK // 16), grp, i32(0))

    def outer(io, carry):
        for b in range(2):
            i = io * i32(2) + i32(b)

            @pl.when(io > i32(0))
            def _():
                wait_scatter(i - i32(2), b)
            fill(i, b)
            scatter(i, b)
        return carry
    lax.fori_loop(i32(0), i32(NCHUNK // 2), outer, i32(0))
    wait_scatter(i32(NCHUNK - 2), 0)
    wait_scatter(i32(NCHUNK - 1), 1)
    plsc.subcore_barrier()
    pltpu.sync_copy(acc.at[pl.ds(sid * i32(RPT), RPT)],
                    out_hbm.at[cid, pl.ds(sid * i32(RPT), RPT)])


def _deg_call(col3, ew3, zero16):
    return pl.kernel(
        _deg_body,
        out_type=jax.ShapeDtypeStruct((NC, N, 16), f32),
        mesh=_sc_mesh(),
        scratch_types=[
            pltpu.VMEM((NCHUNK, CHUNK), jnp.int32),
            pltpu.VMEM((NCHUNK, CHUNK), f32),
            pltpu.VMEM((2, CHUNK, 16), f32),
            pltpu.SemaphoreType.DMA,
            pltpu.SemaphoreType.DMA,
            pltpu.VMEM_SHARED((N, 16), f32),
        ],
        compiler_params=_SC_PARAMS,
    )(col3, ew3, zero16)


def _conv_body(D, hs_hbm, row_hbm, col_hbm, ew_hbm, zero_hbm, out_hbm,
               rowv, colv, ewv, buf, g0, g1, g2, g3, s0, s1, s2, s3, acc):
    cid, sid, wid = _wid()
    pltpu.sync_copy(zero_hbm.at[pl.ds(sid * i32(RPT), RPT)],
                    acc.at[pl.ds(sid * i32(RPT), RPT)])
    pltpu.sync_copy(row_hbm.at[wid], rowv)
    pltpu.sync_copy(col_hbm.at[wid], colv)
    pltpu.sync_copy(ew_hbm.at[wid], ewv)
    plsc.subcore_barrier()
    nvpr = D // 16
    gsems = (g0, g1, g2, g3)
    ssems = (s0, s1, s2, s3)

    def gather(i, b):
        pltpu.async_copy(hs_hbm.at[rowv.at[i]], buf.at[i32(b)], gsems[b])

    def wait_gather(i, b):
        pltpu.make_async_copy(hs_hbm.at[rowv.at[i]], buf.at[i32(b)],
                              gsems[b]).wait()

    def scatter(i, b):
        pltpu.async_copy(buf.at[i32(b)], acc.at[colv.at[i]], ssems[b], add=True)

    def wait_scatter(i, b):
        pltpu.make_async_copy(buf.at[i32(b)], acc.at[colv.at[i]],
                              ssems[b]).wait()

    def scale(i, b):
        def grp(g, c2):
            ews = ewv[i, pl.ds(g * i32(16), 16)]
            for r in range(16):
                rr = g * i32(16) + i32(r)
                s = ews[r]
                for j in range(nvpr):
                    buf[b, rr, pl.ds(j * 16, 16)] = (
                        buf[b, rr, pl.ds(j * 16, 16)] * s)
            return c2
        lax.fori_loop(i32(0), i32(CHUNK // 16), grp, i32(0))

    gather(i32(0), 0)
    gather(i32(1), 1)
    gather(i32(2), 2)

    def outer(io, carry):
        for b in range(4):
            i = io * i32(4) + i32(b)
            wait_gather(i, b)
            scale(i, b)
            scatter(i, b)
            b1 = (b - 1) % 4

            @pl.when(i > i32(0))
            def _():
                wait_scatter(i - i32(1), b1)

            @pl.when(i < i32(NCHUNK - 3))
            def _():
                gather(i + i32(3), b1)
        return carry
    lax.fori_loop(i32(0), i32(NCHUNK // 4), outer, i32(0))
    wait_scatter(i32(NCHUNK - 1), (NCHUNK - 1) % 4)
    plsc.subcore_barrier()
    pltpu.sync_copy(acc.at[pl.ds(sid * i32(RPT), RPT)],
                    out_hbm.at[cid, pl.ds(sid * i32(RPT), RPT)])


def _conv_call(D, hs, row3, col3, ew3, zeroD):
    return pl.kernel(
        functools.partial(_conv_body, D),
        out_type=jax.ShapeDtypeStruct((NC, N, D), f32),
        mesh=_sc_mesh(),
        scratch_types=[
            pltpu.VMEM((NCHUNK, CHUNK), jnp.int32),
            pltpu.VMEM((NCHUNK, CHUNK), jnp.int32),
            pltpu.VMEM((NCHUNK, CHUNK), f32),
            pltpu.VMEM((4, CHUNK, D), f32),
            pltpu.SemaphoreType.DMA,
            pltpu.SemaphoreType.DMA,
            pltpu.SemaphoreType.DMA,
            pltpu.SemaphoreType.DMA,
            pltpu.SemaphoreType.DMA,
            pltpu.SemaphoreType.DMA,
            pltpu.SemaphoreType.DMA,
            pltpu.SemaphoreType.DMA,
            pltpu.VMEM_SHARED((N, D), f32),
        ],
        compiler_params=_SC_PARAMS,
    )(hs, row3, col3, ew3, zeroD)


def _dis(degp_ref):
    deg = (degp_ref[0] + degp_ref[1])[:, 0:1] + 1.0
    return lax.rsqrt(deg)


def _tc1_body(x_ref, w1_ref, degp_ref, hs1_ref):
    hs1_ref[...] = _dis(degp_ref) * jnp.dot(
        x_ref[...], w1_ref[...], preferred_element_type=f32)


def _tc2_body(degp_ref, acc1_ref, hs1_ref, w2_ref, b1_ref, hs2_ref):
    dis = _dis(degp_ref)
    h1 = jax.nn.relu(dis * (acc1_ref[0] + acc1_ref[1] + hs1_ref[...])
                     + b1_ref[...])
    hs2_ref[...] = dis * jnp.dot(h1, w2_ref[...], preferred_element_type=f32)


def _tc3_body(degp_ref, acc2_ref, hs2_ref, b2_ref, x_ref, wd_ref, hs3_ref):
    dis = _dis(degp_ref)
    z = jnp.tanh(dis * (acc2_ref[0] + acc2_ref[1] + hs2_ref[...])
                 + b2_ref[...])
    enc = jnp.concatenate([z, z, z, z], axis=1) + x_ref[...]
    hs3_ref[...] = dis * jnp.dot(enc, wd_ref[...], preferred_element_type=f32)


def _tc4_body(degp_ref, acc3_ref, hs3_ref, bd_ref, wf_ref, bf_ref, out_ref):
    dis = _dis(degp_ref)
    d = jax.nn.relu(dis * (acc3_ref[0] + acc3_ref[1] + hs3_ref[...])
                    + bd_ref[...])
    s = jnp.sum(d * wf_ref[...], keepdims=True).reshape(1, 1) + bf_ref[...]
    out_ref[...] = jax.nn.sigmoid(s)


def _tc_call(body, out_shape, *args):
    return pl.pallas_call(
        body, out_shape=jax.ShapeDtypeStruct(out_shape, f32))(*args)


def kernel(features, edge_index, edge_attr, W1, b1, W2, b2, Wd, bd, Wf, bf):
    x = features.astype(f32)
    ew3 = edge_attr.astype(f32).reshape(NW, NCHUNK, CHUNK)
    row3 = edge_index[0].astype(jnp.int32).reshape(NW, NCHUNK, CHUNK)
    col3 = edge_index[1].astype(jnp.int32).reshape(NW, NCHUNK, CHUNK)
    w1 = W1.astype(f32)
    b1f = b1.astype(f32).reshape(1, -1)
    w2 = W2.astype(f32)
    b2f = b2.astype(f32).reshape(1, -1)
    wd = jnp.zeros((x.shape[1], 16), f32).at[:, :2].set(Wd.astype(f32))
    bdp = jnp.zeros((1, 16), f32).at[0, :2].set(bd.astype(f32))
    wf = jnp.zeros((N, 16), f32).at[:, :2].set(Wf.astype(f32).reshape(N, 2))
    bff = bf.astype(f32).reshape(1, 1)

    zero16 = jnp.zeros((N, 16), f32)
    zero64 = jnp.zeros((N, 64), f32)
    zero32 = jnp.zeros((N, 32), f32)

    degp = _deg_call(col3, ew3, zero16)
    hs1 = _tc_call(_tc1_body, (N, 64), x, w1, degp)
    acc1 = _conv_call(64, hs1, row3, col3, ew3, zero64)
    hs2 = _tc_call(_tc2_body, (N, 32), degp, acc1, hs1, w2, b1f)
    acc2 = _conv_call(32, hs2, row3, col3, ew3, zero32)
    hs3 = _tc_call(_tc3_body, (N, 16), degp, acc2, hs2, b2f, x, wd)
    acc3 = _conv_call(16, hs3, row3, col3, ew3, zero16)
    out = _tc_call(_tc4_body, (1, 1), degp, acc3, hs3, bdp, wf, bff)
    return out.reshape(()).astype(jnp.float64)


# speedup vs baseline: 382.4188x; 1.1211x over previous
import functools

import jax
import jax.numpy as jnp
from jax import lax
from jax.experimental import pallas as pl
from jax.experimental.pallas import tpu as pltpu
from jax.experimental.pallas import tpu_sc as plsc

N = 4096
E = 131072
NC, NS = 2, 16
NW = NC * NS
EPW = E // NW
CHUNK = 128
NCHUNK = EPW // CHUNK
RPT = N // NS

f32 = jnp.float32
i32 = jnp.int32

_SC_PARAMS = pltpu.CompilerParams(use_tc_tiling_on_sc=False)


def _wid():
    cid = jnp.int32(lax.axis_index("c"))
    sid = jnp.int32(lax.axis_index("s"))
    return cid, sid, sid * jnp.int32(NC) + cid


def _sc_mesh():
    return plsc.VectorSubcoreMesh(
        core_axis_name="c", subcore_axis_name="s",
        num_cores=NC, num_subcores=NS)


def _deg_body(col_hbm, ew_hbm, zero_hbm, out_hbm, colv, ewv, buf, s0, s1,
              acc):
    cid, sid, wid = _wid()
    pltpu.sync_copy(zero_hbm.at[pl.ds(sid * i32(RPT), RPT)],
                    acc.at[pl.ds(sid * i32(RPT), RPT)])
    pltpu.sync_copy(col_hbm.at[wid], colv)
    pltpu.sync_copy(ew_hbm.at[wid], ewv)
    plsc.subcore_barrier()
    sems = (s0, s1)

    def scatter(i, b):
        pltpu.async_copy(buf.at[i32(b)], acc.at[colv.at[i]], sems[b], add=True)

    def wait_scatter(i, b):
        pltpu.make_async_copy(buf.at[i32(b)], acc.at[colv.at[i]], sems[b]).wait()

    def fill(i, b):
        def grp(g, c2):
            ews = ewv[i, pl.ds(g * i32(16), 16)]
            for r in range(16):
                buf[b, g * i32(16) + i32(r), :] = jnp.full((16,), ews[r],
                                                           dtype=f32)
            return c2
        lax.fori_loop(i32(0), i32(CHUNK // 16), grp, i32(0))

    def outer(io, carry):
        for b in range(2):
            i = io * i32(2) + i32(b)

            @pl.when(io > i32(0))
            def _():
                wait_scatter(i - i32(2), b)
            fill(i, b)
            scatter(i, b)
        return carry
    lax.fori_loop(i32(0), i32(NCHUNK // 2), outer, i32(0))
    wait_scatter(i32(NCHUNK - 2), 0)
    wait_scatter(i32(NCHUNK - 1), 1)
    plsc.subcore_barrier()
    pltpu.sync_copy(acc.at[pl.ds(sid * i32(RPT), RPT)],
                    out_hbm.at[cid, pl.ds(sid * i32(RPT), RPT)])


def _deg_call(col3, ew3, zero16):
    return pl.kernel(
        _deg_body,
        out_type=jax.ShapeDtypeStruct((NC, N, 16), f32),
        mesh=_sc_mesh(),
        scratch_types=[
            pltpu.VMEM((NCHUNK, CHUNK), jnp.int32),
            pltpu.VMEM((NCHUNK, CHUNK), f32),
            pltpu.VMEM((2, CHUNK, 16), f32),
            pltpu.SemaphoreType.DMA,
            pltpu.SemaphoreType.DMA,
            pltpu.VMEM_SHARED((N, 16), f32),
        ],
        compiler_params=_SC_PARAMS,
    )(col3, ew3, zero16)


def _conv_body(D, hs_hbm, row_hbm, col_hbm, ew_hbm, zero_hbm, out_hbm,
               rowv, colv, ewv, buf, g0, g1, g2, g3, s0, s1, s2, s3, acc):
    cid, sid, wid = _wid()
    pltpu.sync_copy(zero_hbm.at[pl.ds(sid * i32(RPT), RPT)],
                    acc.at[pl.ds(sid * i32(RPT), RPT)])
    pltpu.sync_copy(row_hbm.at[wid], rowv)
    pltpu.sync_copy(col_hbm.at[wid], colv)
    pltpu.sync_copy(ew_hbm.at[wid], ewv)
    plsc.subcore_barrier()
    nvpr = D // 16
    cps = 128 // nvpr // 16
    nstep = NCHUNK // cps
    gsems = (g0, g1, g2, g3)
    ssems = (s0, s1, s2, s3)

    def gather(t, b):
        for cc in range(cps):
            c = t * i32(cps) + i32(cc)
            pltpu.async_copy(hs_hbm.at[rowv.at[c]],
                             buf.at[i32(b), pl.ds(cc * CHUNK, CHUNK)],
                             gsems[b])

    def wait_gather(t, b):
        for cc in range(cps):
            c = t * i32(cps) + i32(cc)
            pltpu.make_async_copy(hs_hbm.at[rowv.at[c]],
                                  buf.at[i32(b), pl.ds(cc * CHUNK, CHUNK)],
                                  gsems[b]).wait()

    def scatter(t, b):
        for cc in range(cps):
            c = t * i32(cps) + i32(cc)
            pltpu.async_copy(buf.at[i32(b), pl.ds(cc * CHUNK, CHUNK)],
                             acc.at[colv.at[c]], ssems[b], add=True)

    def wait_scatter(t, b):
        for cc in range(cps):
            c = t * i32(cps) + i32(cc)
            pltpu.make_async_copy(buf.at[i32(b), pl.ds(cc * CHUNK, CHUNK)],
                                  acc.at[colv.at[c]], ssems[b]).wait()

    def scale(t, b):
        def grp(g, c2):
            c = t * i32(cps) + g // i32(CHUNK // 16)
            gg = g % i32(CHUNK // 16)
            ews = ewv[c, pl.ds(gg * i32(16), 16)]
            for r in range(16):
                rr = g * i32(16) + i32(r)
                s = ews[r]
                for j in range(nvpr):
                    buf[b, rr, pl.ds(j * 16, 16)] = (
                        buf[b, rr, pl.ds(j * 16, 16)] * s)
            return c2
        lax.fori_loop(i32(0), i32(cps * CHUNK // 16), grp, i32(0))

    gather(i32(0), 0)
    gather(i32(1), 1)
    gather(i32(2), 2)

    def step(t, b):
        wait_gather(t, b)
        scale(t, b)
        scatter(t, b)
        b1 = (b - 1) % 4

        @pl.when(t > i32(0))
        def _():
            wait_scatter(t - i32(1), b1)

        @pl.when(t < i32(nstep - 3))
        def _():
            gather(t + i32(3), b1)

    if nstep == 4:
        for b in range(4):
            step(i32(b), b)
    else:
        def outer(io, carry):
            for b in range(4):
                step(io * i32(4) + i32(b), b)
            return carry
        lax.fori_loop(i32(0), i32(nstep // 4), outer, i32(0))
    wait_scatter(i32(nstep - 1), (nstep - 1) % 4)
    plsc.subcore_barrier()
    pltpu.sync_copy(acc.at[pl.ds(sid * i32(RPT), RPT)],
                    out_hbm.at[cid, pl.ds(sid * i32(RPT), RPT)])


def _conv_call(D, hs, row3, col3, ew3, zeroD):
    return pl.kernel(
        functools.partial(_conv_body, D),
        out_type=jax.ShapeDtypeStruct((NC, N, D), f32),
        mesh=_sc_mesh(),
        scratch_types=[
            pltpu.VMEM((NCHUNK, CHUNK), jnp.int32),
            pltpu.VMEM((NCHUNK, CHUNK), jnp.int32),
            pltpu.VMEM((NCHUNK, CHUNK), f32),
            pltpu.VMEM((4, CHUNK * (128 // (D // 16) // 16), D), f32),
            pltpu.SemaphoreType.DMA,
            pltpu.SemaphoreType.DMA,
            pltpu.SemaphoreType.DMA,
            pltpu.SemaphoreType.DMA,
            pltpu.SemaphoreType.DMA,
            pltpu.SemaphoreType.DMA,
            pltpu.SemaphoreType.DMA,
            pltpu.SemaphoreType.DMA,
            pltpu.VMEM_SHARED((N, D), f32),
        ],
        compiler_params=_SC_PARAMS,
    )(hs, row3, col3, ew3, zeroD)


def _dis(degp_ref):
    deg = (degp_ref[0] + degp_ref[1])[:, 0:1] + 1.0
    return lax.rsqrt(deg)


def _tc1_body(x_ref, w1_ref, degp_ref, hs1_ref):
    hs1_ref[...] = _dis(degp_ref) * jnp.dot(
        x_ref[...], w1_ref[...], preferred_element_type=f32)


def _tc2_body(degp_ref, acc1_ref, hs1_ref, w2_ref, b1_ref, hs2_ref):
    dis = _dis(degp_ref)
    h1 = jax.nn.relu(dis * (acc1_ref[0] + acc1_ref[1] + hs1_ref[...])
                     + b1_ref[...])
    hs2_ref[...] = dis * jnp.dot(h1, w2_ref[...], preferred_element_type=f32)


def _tc3_body(degp_ref, acc2_ref, hs2_ref, b2_ref, x_ref, wd_ref, hs3_ref):
    dis = _dis(degp_ref)
    z = jnp.tanh(dis * (acc2_ref[0] + acc2_ref[1] + hs2_ref[...])
                 + b2_ref[...])
    enc = jnp.concatenate([z, z, z, z], axis=1) + x_ref[...]
    hs3_ref[...] = dis * jnp.dot(enc, wd_ref[...], preferred_element_type=f32)


def _tc4_body(degp_ref, acc3_ref, hs3_ref, bd_ref, wf_ref, bf_ref, out_ref):
    dis = _dis(degp_ref)
    d = jax.nn.relu(dis * (acc3_ref[0] + acc3_ref[1] + hs3_ref[...])
                    + bd_ref[...])
    s = jnp.sum(d * wf_ref[...], keepdims=True).reshape(1, 1) + bf_ref[...]
    out_ref[...] = jax.nn.sigmoid(s)


def _tc_call(body, out_shape, *args):
    return pl.pallas_call(
        body, out_shape=jax.ShapeDtypeStruct(out_shape, f32))(*args)


def kernel(features, edge_index, edge_attr, W1, b1, W2, b2, Wd, bd, Wf, bf):
    x = features.astype(f32)
    ew3 = edge_attr.astype(f32).reshape(NW, NCHUNK, CHUNK)
    row3 = edge_index[0].astype(jnp.int32).reshape(NW, NCHUNK, CHUNK)
    col3 = edge_index[1].astype(jnp.int32).reshape(NW, NCHUNK, CHUNK)
    w1 = W1.astype(f32)
    b1f = b1.astype(f32).reshape(1, -1)
    w2 = W2.astype(f32)
    b2f = b2.astype(f32).reshape(1, -1)
    wd = jnp.zeros((x.shape[1], 16), f32).at[:, :2].set(Wd.astype(f32))
    bdp = jnp.zeros((1, 16), f32).at[0, :2].set(bd.astype(f32))
    wf = jnp.zeros((N, 16), f32).at[:, :2].set(Wf.astype(f32).reshape(N, 2))
    bff = bf.astype(f32).reshape(1, 1)

    zero16 = jnp.zeros((N, 16), f32)
    zero64 = jnp.zeros((N, 64), f32)
    zero32 = jnp.zeros((N, 32), f32)

    degp = _deg_call(col3, ew3, zero16)
    hs1 = _tc_call(_tc1_body, (N, 64), x, w1, degp)
    acc1 = _conv_call(64, hs1, row3, col3, ew3, zero64)
    hs2 = _tc_call(_tc2_body, (N, 32), degp, acc1, hs1, w2, b1f)
    acc2 = _conv_call(32, hs2, row3, col3, ew3, zero32)
    hs3 = _tc_call(_tc3_body, (N, 16), degp, acc2, hs2, b2f, x, wd)
    acc3 = _conv_call(16, hs3, row3, col3, ew3, zero16)
    out = _tc_call(_tc4_body, (1, 1), degp, acc3, hs3, bdp, wf, bff)
    return out.reshape(()).astype(jnp.float64)


# speedup vs baseline: 410.2046x; 1.0727x over previous
import functools

import jax
import jax.numpy as jnp
from jax import lax
from jax.experimental import pallas as pl
from jax.experimental.pallas import tpu as pltpu
from jax.experimental.pallas import tpu_sc as plsc

N = 4096
E = 131072
NC, NS = 2, 16
NW = NC * NS
EPW = E // NW
CHUNK = 128
NCHUNK = EPW // CHUNK
RPT = N // NS

f32 = jnp.float32
i32 = jnp.int32

_SC_PARAMS = pltpu.CompilerParams(use_tc_tiling_on_sc=False,
                                  needs_layout_passes=False)


def _wid():
    cid = jnp.int32(lax.axis_index("c"))
    sid = jnp.int32(lax.axis_index("s"))
    return cid, sid, sid * jnp.int32(NC) + cid


def _sc_mesh():
    return plsc.VectorSubcoreMesh(
        core_axis_name="c", subcore_axis_name="s",
        num_cores=NC, num_subcores=NS)


def _rsqrt_sc(x):
    i = plsc.bitcast(x, jnp.int32)
    i = jnp.full((16,), 0x5F3759DF, dtype=jnp.int32) - (
        lax.shift_right_logical(i, jnp.full((16,), 1, dtype=jnp.int32)))
    y = plsc.bitcast(i, f32)
    for _ in range(3):
        y = y * (1.5 - 0.5 * x * y * y)
    return y


def _conv1_body(hs_hbm, row_hbm, col_hbm, ew_hbm, zero_hbm, arange_hbm,
                out_hbm, dis_hbm,
                rowv, colv, ewv, colv2, ewv2, idxv, degv, disv, buf,
                g0, g1, g2, g3, s0, s1, s2, s3, acc, degacc):
    cid, sid, wid = _wid()
    owid = sid * i32(NC) + (i32(1) - cid)
    D = 64
    nvpr = D // 16
    cps = 2
    nstep = NCHUNK // cps
    gsems = (g0, g1, g2, g3)
    ssems = (s0, s1, s2, s3)

    pltpu.sync_copy(zero_hbm.at[pl.ds(sid * i32(RPT), RPT)],
                    acc.at[pl.ds(sid * i32(RPT), RPT)])
    pltpu.sync_copy(row_hbm.at[wid], rowv)
    pltpu.sync_copy(col_hbm.at[wid], colv)
    pltpu.sync_copy(ew_hbm.at[wid], ewv)
    pltpu.sync_copy(col_hbm.at[owid], colv2)
    pltpu.sync_copy(ew_hbm.at[owid], ewv2)
    pltpu.sync_copy(arange_hbm, idxv)

    def gather(t, b):
        for cc in range(cps):
            c = t * i32(cps) + i32(cc)
            pltpu.async_copy(hs_hbm.at[rowv.at[c]],
                             buf.at[i32(b), pl.ds(cc * CHUNK, CHUNK)],
                             gsems[b])

    def wait_gather(t, b):
        for cc in range(cps):
            c = t * i32(cps) + i32(cc)
            pltpu.make_async_copy(hs_hbm.at[rowv.at[c]],
                                  buf.at[i32(b), pl.ds(cc * CHUNK, CHUNK)],
                                  gsems[b]).wait()

    def scatter(t, b):
        for cc in range(cps):
            c = t * i32(cps) + i32(cc)
            pltpu.async_copy(buf.at[i32(b), pl.ds(cc * CHUNK, CHUNK)],
                             acc.at[colv.at[c]], ssems[b], add=True)

    def wait_scatter(t, b):
        for cc in range(cps):
            c = t * i32(cps) + i32(cc)
            pltpu.make_async_copy(buf.at[i32(b), pl.ds(cc * CHUNK, CHUNK)],
                                  acc.at[colv.at[c]], ssems[b]).wait()

    gather(i32(0), 0)
    gather(i32(1), 1)
    gather(i32(2), 2)

    def zrow(g, c2):
        for r in range(16):
            degv[g * i32(16) + i32(r), :] = jnp.zeros((16,), f32)
        return c2
    lax.fori_loop(i32(0), i32(16), zrow, i32(0))
    pltpu.sync_copy(degv.at[pl.ds(i32(0), 16)],
                    degacc.at[pl.ds(sid * i32(16), 16)])
    plsc.subcore_barrier()

    mask15 = jnp.full((16,), 15, dtype=jnp.int32)
    four = jnp.full((16,), 4, dtype=jnp.int32)

    def dacc(cv, ev):
        def grp(q, c2):
            c = lax.shift_right_logical(q, i32(3))
            g = q & i32(7)
            cols = cv[c, pl.ds(g * i32(16), 16)]
            ews = ev[c, pl.ds(g * i32(16), 16)]
            plsc.addupdate_scatter(
                degv, [lax.shift_right_logical(cols, four), cols & mask15],
                ews)
            return c2
        lax.fori_loop(i32(0), i32(NCHUNK * CHUNK // 16), grp, i32(0))
    dacc(colv, ewv)
    dacc(colv2, ewv2)
    pltpu.sync_copy(degv.at[pl.ds(i32(0), 128)],
                    degacc.at[idxv.at[i32(0)]], add=True)
    pltpu.sync_copy(degv.at[pl.ds(i32(128), 128)],
                    degacc.at[idxv.at[i32(1)]], add=True)
    plsc.subcore_barrier()
    pltpu.sync_copy(degacc, disv)

    def drow(g, c2):
        for r in range(16):
            rr = g * i32(16) + i32(r)
            disv[rr, :] = _rsqrt_sc(disv[rr, :] + 1.0)
        return c2
    lax.fori_loop(i32(0), i32(16), drow, i32(0))

    @pl.when((cid == i32(0)) & (sid == i32(0)))
    def _():
        pltpu.sync_copy(disv, dis_hbm)

    def scale(t, b):
        def grp(g, c2):
            c = t * i32(cps) + lax.shift_right_logical(g, i32(3))
            gg = g & i32(7)
            ews = ewv[c, pl.ds(gg * i32(16), 16)]
            rows = rowv[c, pl.ds(gg * i32(16), 16)]
            disr = plsc.load_gather(
                disv, [lax.shift_right_logical(rows, four), rows & mask15])
            sv = ews * disr
            for r in range(16):
                rr = g * i32(16) + i32(r)
                s = sv[r]
                for j in range(nvpr):
                    buf[b, rr, pl.ds(j * 16, 16)] = (
                        buf[b, rr, pl.ds(j * 16, 16)] * s)
            return c2
        lax.fori_loop(i32(0), i32(cps * CHUNK // 16), grp, i32(0))

    def step(t, b):
        wait_gather(t, b)
        scale(t, b)
        scatter(t, b)
        b1 = (b - 1) % 4

        @pl.when(t > i32(0))
        def _():
            wait_scatter(t - i32(1), b1)

        @pl.when(t < i32(nstep - 3))
        def _():
            gather(t + i32(3), b1)

    def outer(io, carry):
        for b in range(4):
            step(io * i32(4) + i32(b), b)
        return carry
    lax.fori_loop(i32(0), i32(nstep // 4), outer, i32(0))
    wait_scatter(i32(nstep - 1), (nstep - 1) % 4)
    plsc.subcore_barrier()
    pltpu.sync_copy(acc.at[pl.ds(sid * i32(RPT), RPT)],
                    out_hbm.at[cid, pl.ds(sid * i32(RPT), RPT)])


def _conv1_call(hs, row3, col3, ew3, zero64, arange2):
    return pl.kernel(
        _conv1_body,
        out_type=(jax.ShapeDtypeStruct((NC, N, 64), f32),
                  jax.ShapeDtypeStruct((RPT, 16), f32)),
        mesh=_sc_mesh(),
        scratch_types=[
            pltpu.VMEM((NCHUNK, CHUNK), jnp.int32),
            pltpu.VMEM((NCHUNK, CHUNK), jnp.int32),
            pltpu.VMEM((NCHUNK, CHUNK), f32),
            pltpu.VMEM((NCHUNK, CHUNK), jnp.int32),
            pltpu.VMEM((NCHUNK, CHUNK), f32),
            pltpu.VMEM((2, 128), jnp.int32),
            pltpu.VMEM((RPT, 16), f32),
            pltpu.VMEM((RPT, 16), f32),
            pltpu.VMEM((4, CHUNK * 2, 64), f32),
            pltpu.SemaphoreType.DMA,
            pltpu.SemaphoreType.DMA,
            pltpu.SemaphoreType.DMA,
            pltpu.SemaphoreType.DMA,
            pltpu.SemaphoreType.DMA,
            pltpu.SemaphoreType.DMA,
            pltpu.SemaphoreType.DMA,
            pltpu.SemaphoreType.DMA,
            pltpu.VMEM_SHARED((N, 64), f32),
            pltpu.VMEM_SHARED((RPT, 16), f32),
        ],
        compiler_params=_SC_PARAMS,
    )(hs, row3, col3, ew3, zero64, arange2)


def _conv_body(D, hs_hbm, row_hbm, col_hbm, ew_hbm, zero_hbm, out_hbm,
               rowv, colv, ewv, buf, g0, g1, g2, g3, s0, s1, s2, s3, acc):
    cid, sid, wid = _wid()
    pltpu.sync_copy(zero_hbm.at[pl.ds(sid * i32(RPT), RPT)],
                    acc.at[pl.ds(sid * i32(RPT), RPT)])
    pltpu.sync_copy(row_hbm.at[wid], rowv)
    pltpu.sync_copy(col_hbm.at[wid], colv)
    pltpu.sync_copy(ew_hbm.at[wid], ewv)
    plsc.subcore_barrier()
    nvpr = D // 16
    cps = 128 // nvpr // 16
    nstep = NCHUNK // cps
    gsems = (g0, g1, g2, g3)
    ssems = (s0, s1, s2, s3)

    def gather(t, b):
        for cc in range(cps):
            c = t * i32(cps) + i32(cc)
            pltpu.async_copy(hs_hbm.at[rowv.at[c]],
                             buf.at[i32(b), pl.ds(cc * CHUNK, CHUNK)],
                             gsems[b])

    def wait_gather(t, b):
        for cc in range(cps):
            c = t * i32(cps) + i32(cc)
            pltpu.make_async_copy(hs_hbm.at[rowv.at[c]],
                                  buf.at[i32(b), pl.ds(cc * CHUNK, CHUNK)],
                                  gsems[b]).wait()

    def scatter(t, b):
        for cc in range(cps):
            c = t * i32(cps) + i32(cc)
            pltpu.async_copy(buf.at[i32(b), pl.ds(cc * CHUNK, CHUNK)],
                             acc.at[colv.at[c]], ssems[b], add=True)

    def wait_scatter(t, b):
        for cc in range(cps):
            c = t * i32(cps) + i32(cc)
            pltpu.make_async_copy(buf.at[i32(b), pl.ds(cc * CHUNK, CHUNK)],
                                  acc.at[colv.at[c]], ssems[b]).wait()

    def scale(t, b):
        def grp(g, c2):
            c = t * i32(cps) + g // i32(CHUNK // 16)
            gg = g % i32(CHUNK // 16)
            ews = ewv[c, pl.ds(gg * i32(16), 16)]
            for r in range(16):
                rr = g * i32(16) + i32(r)
                s = ews[r]
                for j in range(nvpr):
                    buf[b, rr, pl.ds(j * 16, 16)] = (
                        buf[b, rr, pl.ds(j * 16, 16)] * s)
            return c2
        lax.fori_loop(i32(0), i32(cps * CHUNK // 16), grp, i32(0))

    gather(i32(0), 0)
    gather(i32(1), 1)
    gather(i32(2), 2)

    def step(t, b):
        wait_gather(t, b)
        scale(t, b)
        scatter(t, b)
        b1 = (b - 1) % 4

        @pl.when(t > i32(0))
        def _():
            wait_scatter(t - i32(1), b1)

        @pl.when(t < i32(nstep - 3))
        def _():
            gather(t + i32(3), b1)

    if nstep == 4:
        for b in range(4):
            step(i32(b), b)
    else:
        def outer(io, carry):
            for b in range(4):
                step(io * i32(4) + i32(b), b)
            return carry
        lax.fori_loop(i32(0), i32(nstep // 4), outer, i32(0))
    wait_scatter(i32(nstep - 1), (nstep - 1) % 4)
    plsc.subcore_barrier()
    pltpu.sync_copy(acc.at[pl.ds(sid * i32(RPT), RPT)],
                    out_hbm.at[cid, pl.ds(sid * i32(RPT), RPT)])


def _conv_call(D, hs, row3, col3, ew3, zeroD):
    return pl.kernel(
        functools.partial(_conv_body, D),
        out_type=jax.ShapeDtypeStruct((NC, N, D), f32),
        mesh=_sc_mesh(),
        scratch_types=[
            pltpu.VMEM((NCHUNK, CHUNK), jnp.int32),
            pltpu.VMEM((NCHUNK, CHUNK), jnp.int32),
            pltpu.VMEM((NCHUNK, CHUNK), f32),
            pltpu.VMEM((4, CHUNK * (128 // (D // 16) // 16), D), f32),
            pltpu.SemaphoreType.DMA,
            pltpu.SemaphoreType.DMA,
            pltpu.SemaphoreType.DMA,
            pltpu.SemaphoreType.DMA,
            pltpu.SemaphoreType.DMA,
            pltpu.SemaphoreType.DMA,
            pltpu.SemaphoreType.DMA,
            pltpu.SemaphoreType.DMA,
            pltpu.VMEM_SHARED((N, D), f32),
        ],
        compiler_params=_SC_PARAMS,
    )(hs, row3, col3, ew3, zeroD)


def _tc1_body(x_ref, w1_ref, h0_ref):
    h0_ref[...] = jnp.dot(x_ref[...], w1_ref[...],
                          preferred_element_type=f32)


def _tc2_body(dis_ref, acc1_ref, h0_ref, w2_ref, b1_ref, hs2_ref):
    dis = dis_ref[...]
    h1 = jax.nn.relu(dis * (acc1_ref[0] + acc1_ref[1] + dis * h0_ref[...])
                     + b1_ref[...])
    hs2_ref[...] = dis * jnp.dot(h1, w2_ref[...], preferred_element_type=f32)


def _tc3_body(dis_ref, acc2_ref, hs2_ref, b2_ref, x_ref, wd_ref, hs3_ref):
    dis = dis_ref[...]
    z = jnp.tanh(dis * (acc2_ref[0] + acc2_ref[1] + hs2_ref[...])
                 + b2_ref[...])
    enc = jnp.concatenate([z, z, z, z], axis=1) + x_ref[...]
    hs3_ref[...] = dis * jnp.dot(enc, wd_ref[...], preferred_element_type=f32)


def _tc4_body(dis_ref, acc3_ref, hs3_ref, bd_ref, wf_ref, bf_ref, out_ref):
    dis = dis_ref[...]
    d = jax.nn.relu(dis * (acc3_ref[0] + acc3_ref[1] + hs3_ref[...])
                    + bd_ref[...])
    s = jnp.sum(d * wf_ref[...], keepdims=True).reshape(1, 1) + bf_ref[...]
    out_ref[...] = jax.nn.sigmoid(s)


def _tc_call(body, out_shape, *args):
    return pl.pallas_call(
        body, out_shape=jax.ShapeDtypeStruct(out_shape, f32))(*args)


def kernel(features, edge_index, edge_attr, W1, b1, W2, b2, Wd, bd, Wf, bf):
    x = features.astype(f32)
    ew3 = edge_attr.astype(f32).reshape(NW, NCHUNK, CHUNK)
    row3 = edge_index[0].astype(jnp.int32).reshape(NW, NCHUNK, CHUNK)
    col3 = edge_index[1].astype(jnp.int32).reshape(NW, NCHUNK, CHUNK)
    w1 = W1.astype(f32)
    b1f = b1.astype(f32).reshape(1, -1)
    w2 = W2.astype(f32)
    b2f = b2.astype(f32).reshape(1, -1)
    wd = jnp.zeros((x.shape[1], 16), f32).at[:, :2].set(Wd.astype(f32))
    bdp = jnp.zeros((1, 16), f32).at[0, :2].set(bd.astype(f32))
    wf = jnp.zeros((N, 16), f32).at[:, :2].set(Wf.astype(f32).reshape(N, 2))
    bff = bf.astype(f32).reshape(1, 1)

    zero16 = jnp.zeros((N, 16), f32)
    zero64 = jnp.zeros((N, 64), f32)
    zero32 = jnp.zeros((N, 32), f32)
    arange2 = jnp.arange(N // 16, dtype=jnp.int32).reshape(2, 128)

    h0 = _tc_call(_tc1_body, (N, 64), x, w1)
    acc1, dis16 = _conv1_call(h0, row3, col3, ew3, zero64, arange2)
    dis = dis16.reshape(N, 1)
    hs2 = _tc_call(_tc2_body, (N, 32), dis, acc1, h0, w2, b1f)
    acc2 = _conv_call(32, hs2, row3, col3, ew3, zero32)
    hs3 = _tc_call(_tc3_body, (N, 16), dis, acc2, hs2, b2f, x, wd)
    acc3 = _conv_call(16, hs3, row3, col3, ew3, zero16)
    out = _tc_call(_tc4_body, (1, 1), dis, acc3, hs3, bdp, wf, bff)
    return out.reshape(()).astype(jnp.float64)


# speedup vs baseline: 432.8756x; 1.0553x over previous
import functools

import jax
import jax.numpy as jnp
from jax import lax
from jax.experimental import pallas as pl
from jax.experimental.pallas import tpu as pltpu
from jax.experimental.pallas import tpu_sc as plsc

N = 4096
E = 131072
NC, NS = 2, 16
NW = NC * NS
EPW = E // NW
CHUNK = 128
NCHUNK = EPW // CHUNK
RPT = N // NS

f32 = jnp.float32
i32 = jnp.int32

_SC_PARAMS = pltpu.CompilerParams(use_tc_tiling_on_sc=False,
                                  needs_layout_passes=False)


def _wid():
    cid = jnp.int32(lax.axis_index("c"))
    sid = jnp.int32(lax.axis_index("s"))
    return cid, sid, sid * jnp.int32(NC) + cid


def _sc_mesh():
    return plsc.VectorSubcoreMesh(
        core_axis_name="c", subcore_axis_name="s",
        num_cores=NC, num_subcores=NS)


def _rsqrt_sc(x):
    i = plsc.bitcast(x, jnp.int32)
    i = jnp.full((16,), 0x5F3759DF, dtype=jnp.int32) - (
        lax.shift_right_logical(i, jnp.full((16,), 1, dtype=jnp.int32)))
    y = plsc.bitcast(i, f32)
    for _ in range(3):
        y = y * (1.5 - 0.5 * x * y * y)
    return y


def _conv1_body(hs_hbm, row_hbm, col_hbm, ew_hbm, zero_hbm, arange_hbm,
                out_hbm, dis_hbm,
                rowv, colv, ewv, colv2, ewv2, idxv, degv, disv, buf,
                g0, g1, g2, g3, s0, s1, s2, s3, acc, degacc):
    cid, sid, wid = _wid()
    owid = sid * i32(NC) + (i32(1) - cid)
    D = 64
    nvpr = D // 16
    cps = 2
    nstep = NCHUNK // cps
    gsems = (g0, g1, g2, g3)
    ssems = (s0, s1, s2, s3)

    dz = pltpu.async_copy(zero_hbm.at[pl.ds(sid * i32(RPT), RPT)],
                          acc.at[pl.ds(sid * i32(RPT), RPT)], s0)
    dr = pltpu.async_copy(row_hbm.at[wid], rowv, s1)
    dc = pltpu.async_copy(col_hbm.at[wid], colv, s2)
    de = pltpu.async_copy(ew_hbm.at[wid], ewv, s3)
    dc2 = pltpu.async_copy(col_hbm.at[owid], colv2, g3)
    de2 = pltpu.async_copy(ew_hbm.at[owid], ewv2, s0)
    dar = pltpu.async_copy(arange_hbm, idxv, s1)

    def gather(t, b):
        for cc in range(cps):
            c = t * i32(cps) + i32(cc)
            pltpu.async_copy(hs_hbm.at[rowv.at[c]],
                             buf.at[i32(b), pl.ds(cc * CHUNK, CHUNK)],
                             gsems[b])

    def wait_gather(t, b):
        for cc in range(cps):
            c = t * i32(cps) + i32(cc)
            pltpu.make_async_copy(hs_hbm.at[rowv.at[c]],
                                  buf.at[i32(b), pl.ds(cc * CHUNK, CHUNK)],
                                  gsems[b]).wait()

    def scatter(t, b):
        for cc in range(cps):
            c = t * i32(cps) + i32(cc)
            pltpu.async_copy(buf.at[i32(b), pl.ds(cc * CHUNK, CHUNK)],
                             acc.at[colv.at[c]], ssems[b], add=True)

    def wait_scatter(t, b):
        for cc in range(cps):
            c = t * i32(cps) + i32(cc)
            pltpu.make_async_copy(buf.at[i32(b), pl.ds(cc * CHUNK, CHUNK)],
                                  acc.at[colv.at[c]], ssems[b]).wait()

    def zrow(g, c2):
        for r in range(16):
            degv[g * i32(16) + i32(r), :] = jnp.zeros((16,), f32)
        return c2
    lax.fori_loop(i32(0), i32(16), zrow, i32(0))

    dr.wait()
    gather(i32(0), 0)
    gather(i32(1), 1)
    gather(i32(2), 2)
    dz.wait()
    dc.wait()
    de.wait()
    dc2.wait()
    de2.wait()
    dar.wait()
    pltpu.sync_copy(degv.at[pl.ds(i32(0), 16)],
                    degacc.at[pl.ds(sid * i32(16), 16)])
    plsc.subcore_barrier()

    mask15 = jnp.full((16,), 15, dtype=jnp.int32)
    four = jnp.full((16,), 4, dtype=jnp.int32)

    def dacc(cv, ev):
        def grp(q, c2):
            c = lax.shift_right_logical(q, i32(3))
            g = q & i32(7)
            cols = cv[c, pl.ds(g * i32(16), 16)]
            ews = ev[c, pl.ds(g * i32(16), 16)]
            plsc.addupdate_scatter(
                degv, [lax.shift_right_logical(cols, four), cols & mask15],
                ews)
            return c2
        lax.fori_loop(i32(0), i32(NCHUNK * CHUNK // 16), grp, i32(0))
    dacc(colv, ewv)
    dacc(colv2, ewv2)
    pltpu.sync_copy(degv.at[pl.ds(i32(0), 128)],
                    degacc.at[idxv.at[i32(0)]], add=True)
    pltpu.sync_copy(degv.at[pl.ds(i32(128), 128)],
                    degacc.at[idxv.at[i32(1)]], add=True)
    plsc.subcore_barrier()
    pltpu.sync_copy(degacc, disv)

    def drow(g, c2):
        for r in range(16):
            rr = g * i32(16) + i32(r)
            disv[rr, :] = _rsqrt_sc(disv[rr, :] + 1.0)
        return c2
    lax.fori_loop(i32(0), i32(16), drow, i32(0))

    @pl.when((cid == i32(0)) & (sid == i32(0)))
    def _():
        pltpu.sync_copy(disv, dis_hbm)

    def scale(t, b):
        def grp(g, c2):
            c = t * i32(cps) + lax.shift_right_logical(g, i32(3))
            gg = g & i32(7)
            ews = ewv[c, pl.ds(gg * i32(16), 16)]
            rows = rowv[c, pl.ds(gg * i32(16), 16)]
            disr = plsc.load_gather(
                disv, [lax.shift_right_logical(rows, four), rows & mask15])
            sv = ews * disr
            for r in range(16):
                rr = g * i32(16) + i32(r)
                s = sv[r]
                for j in range(nvpr):
                    buf[b, rr, pl.ds(j * 16, 16)] = (
                        buf[b, rr, pl.ds(j * 16, 16)] * s)
            return c2
        lax.fori_loop(i32(0), i32(cps * CHUNK // 16), grp, i32(0))

    def step(t, b):
        wait_gather(t, b)
        scale(t, b)
        scatter(t, b)
        b1 = (b - 1) % 4

        @pl.when(t > i32(0))
        def _():
            wait_scatter(t - i32(1), b1)

        @pl.when(t < i32(nstep - 3))
        def _():
            gather(t + i32(3), b1)

    def outer(io, carry):
        for b in range(4):
            step(io * i32(4) + i32(b), b)
        return carry
    lax.fori_loop(i32(0), i32(nstep // 4), outer, i32(0))
    wait_scatter(i32(nstep - 1), (nstep - 1) % 4)
    plsc.subcore_barrier()
    pltpu.sync_copy(acc.at[pl.ds(sid * i32(RPT), RPT)],
                    out_hbm.at[cid, pl.ds(sid * i32(RPT), RPT)])


def _conv1_call(hs, row3, col3, ew3, zero64, arange2):
    return pl.kernel(
        _conv1_body,
        out_type=(jax.ShapeDtypeStruct((NC, N, 64), f32),
                  jax.ShapeDtypeStruct((RPT, 16), f32)),
        mesh=_sc_mesh(),
        scratch_types=[
            pltpu.VMEM((NCHUNK, CHUNK), jnp.int32),
            pltpu.VMEM((NCHUNK, CHUNK), jnp.int32),
            pltpu.VMEM((NCHUNK, CHUNK), f32),
            pltpu.VMEM((NCHUNK, CHUNK), jnp.int32),
            pltpu.VMEM((NCHUNK, CHUNK), f32),
            pltpu.VMEM((2, 128), jnp.int32),
            pltpu.VMEM((RPT, 16), f32),
            pltpu.VMEM((RPT, 16), f32),
            pltpu.VMEM((4, CHUNK * 2, 64), f32),
            pltpu.SemaphoreType.DMA,
            pltpu.SemaphoreType.DMA,
            pltpu.SemaphoreType.DMA,
            pltpu.SemaphoreType.DMA,
            pltpu.SemaphoreType.DMA,
            pltpu.SemaphoreType.DMA,
            pltpu.SemaphoreType.DMA,
            pltpu.SemaphoreType.DMA,
            pltpu.VMEM_SHARED((N, 64), f32),
            pltpu.VMEM_SHARED((RPT, 16), f32),
        ],
        compiler_params=_SC_PARAMS,
    )(hs, row3, col3, ew3, zero64, arange2)


def _conv_body(D, hs_hbm, row_hbm, col_hbm, ew_hbm, zero_hbm, out_hbm,
               rowv, colv, ewv, buf, g0, g1, g2, g3, s0, s1, s2, s3, acc):
    cid, sid, wid = _wid()
    dz = pltpu.async_copy(zero_hbm.at[pl.ds(sid * i32(RPT), RPT)],
                          acc.at[pl.ds(sid * i32(RPT), RPT)], s0)
    dr = pltpu.async_copy(row_hbm.at[wid], rowv, s1)
    dc = pltpu.async_copy(col_hbm.at[wid], colv, s2)
    de = pltpu.async_copy(ew_hbm.at[wid], ewv, s3)
    dr.wait()
    dz.wait()
    dc.wait()
    de.wait()
    plsc.subcore_barrier()
    nvpr = D // 16
    cps = 128 // nvpr // 16
    nstep = NCHUNK // cps
    gsems = (g0, g1, g2, g3)
    ssems = (s0, s1, s2, s3)

    def gather(t, b):
        for cc in range(cps):
            c = t * i32(cps) + i32(cc)
            pltpu.async_copy(hs_hbm.at[rowv.at[c]],
                             buf.at[i32(b), pl.ds(cc * CHUNK, CHUNK)],
                             gsems[b])

    def wait_gather(t, b):
        for cc in range(cps):
            c = t * i32(cps) + i32(cc)
            pltpu.make_async_copy(hs_hbm.at[rowv.at[c]],
                                  buf.at[i32(b), pl.ds(cc * CHUNK, CHUNK)],
                                  gsems[b]).wait()

    def scatter(t, b):
        for cc in range(cps):
            c = t * i32(cps) + i32(cc)
            pltpu.async_copy(buf.at[i32(b), pl.ds(cc * CHUNK, CHUNK)],
                             acc.at[colv.at[c]], ssems[b], add=True)

    def wait_scatter(t, b):
        for cc in range(cps):
            c = t * i32(cps) + i32(cc)
            pltpu.make_async_copy(buf.at[i32(b), pl.ds(cc * CHUNK, CHUNK)],
                                  acc.at[colv.at[c]], ssems[b]).wait()

    def scale(t, b):
        def grp(g, c2):
            c = t * i32(cps) + g // i32(CHUNK // 16)
            gg = g % i32(CHUNK // 16)
            ews = ewv[c, pl.ds(gg * i32(16), 16)]
            for r in range(16):
                rr = g * i32(16) + i32(r)
                s = ews[r]
                for j in range(nvpr):
                    buf[b, rr, pl.ds(j * 16, 16)] = (
                        buf[b, rr, pl.ds(j * 16, 16)] * s)
            return c2
        lax.fori_loop(i32(0), i32(cps * CHUNK // 16), grp, i32(0))

    gather(i32(0), 0)
    gather(i32(1), 1)
    gather(i32(2), 2)

    def step(t, b):
        wait_gather(t, b)
        scale(t, b)
        scatter(t, b)
        b1 = (b - 1) % 4

        @pl.when(t > i32(0))
        def _():
            wait_scatter(t - i32(1), b1)

        @pl.when(t < i32(nstep - 3))
        def _():
            gather(t + i32(3), b1)

    if nstep == 4:
        for b in range(4):
            step(i32(b), b)
    else:
        def outer(io, carry):
            for b in range(4):
                step(io * i32(4) + i32(b), b)
            return carry
        lax.fori_loop(i32(0), i32(nstep // 4), outer, i32(0))
    wait_scatter(i32(nstep - 1), (nstep - 1) % 4)
    plsc.subcore_barrier()
    pltpu.sync_copy(acc.at[pl.ds(sid * i32(RPT), RPT)],
                    out_hbm.at[cid, pl.ds(sid * i32(RPT), RPT)])


def _conv_call(D, hs, row3, col3, ew3, zeroD):
    return pl.kernel(
        functools.partial(_conv_body, D),
        out_type=jax.ShapeDtypeStruct((NC, N, D), f32),
        mesh=_sc_mesh(),
        scratch_types=[
            pltpu.VMEM((NCHUNK, CHUNK), jnp.int32),
            pltpu.VMEM((NCHUNK, CHUNK), jnp.int32),
            pltpu.VMEM((NCHUNK, CHUNK), f32),
            pltpu.VMEM((4, CHUNK * (128 // (D // 16) // 16), D), f32),
            pltpu.SemaphoreType.DMA,
            pltpu.SemaphoreType.DMA,
            pltpu.SemaphoreType.DMA,
            pltpu.SemaphoreType.DMA,
            pltpu.SemaphoreType.DMA,
            pltpu.SemaphoreType.DMA,
            pltpu.SemaphoreType.DMA,
            pltpu.SemaphoreType.DMA,
            pltpu.VMEM_SHARED((N, D), f32),
        ],
        compiler_params=_SC_PARAMS,
    )(hs, row3, col3, ew3, zeroD)


def _tc1_body(x_ref, w1_ref, h0_ref):
    h0_ref[...] = jnp.dot(x_ref[...], w1_ref[...],
                          preferred_element_type=f32)


def _tc2_body(dis_ref, acc1_ref, h0_ref, w2_ref, b1_ref, hs2_ref):
    dis = dis_ref[...]
    h1 = jax.nn.relu(dis * (acc1_ref[0] + acc1_ref[1] + dis * h0_ref[...])
                     + b1_ref[...])
    hs2_ref[...] = dis * jnp.dot(h1, w2_ref[...], preferred_element_type=f32)


def _tc3_body(dis_ref, acc2_ref, hs2_ref, b2_ref, x_ref, wd_ref, hs3_ref):
    dis = dis_ref[...]
    z = jnp.tanh(dis * (acc2_ref[0] + acc2_ref[1] + hs2_ref[...])
                 + b2_ref[...])
    enc = jnp.concatenate([z, z, z, z], axis=1) + x_ref[...]
    hs3_ref[...] = dis * jnp.dot(enc, wd_ref[...], preferred_element_type=f32)


def _tc4_body(dis_ref, acc3_ref, hs3_ref, bd_ref, wf_ref, bf_ref, out_ref):
    dis = dis_ref[...]
    d = jax.nn.relu(dis * (acc3_ref[0] + acc3_ref[1] + hs3_ref[...])
                    + bd_ref[...])
    s = jnp.sum(d * wf_ref[...], keepdims=True).reshape(1, 1) + bf_ref[...]
    out_ref[...] = jax.nn.sigmoid(s)


def _tc_call(body, out_shape, *args):
    return pl.pallas_call(
        body, out_shape=jax.ShapeDtypeStruct(out_shape, f32))(*args)


def kernel(features, edge_index, edge_attr, W1, b1, W2, b2, Wd, bd, Wf, bf):
    x = features.astype(f32)
    ew3 = edge_attr.astype(f32).reshape(NW, NCHUNK, CHUNK)
    row3 = edge_index[0].astype(jnp.int32).reshape(NW, NCHUNK, CHUNK)
    col3 = edge_index[1].astype(jnp.int32).reshape(NW, NCHUNK, CHUNK)
    w1 = W1.astype(f32)
    b1f = b1.astype(f32).reshape(1, -1)
    w2 = W2.astype(f32)
    b2f = b2.astype(f32).reshape(1, -1)
    wd = jnp.zeros((x.shape[1], 16), f32).at[:, :2].set(Wd.astype(f32))
    bdp = jnp.zeros((1, 16), f32).at[0, :2].set(bd.astype(f32))
    wf = jnp.zeros((N, 16), f32).at[:, :2].set(Wf.astype(f32).reshape(N, 2))
    bff = bf.astype(f32).reshape(1, 1)

    zero16 = jnp.zeros((N, 16), f32)
    zero64 = jnp.zeros((N, 64), f32)
    zero32 = jnp.zeros((N, 32), f32)
    arange2 = jnp.arange(N // 16, dtype=jnp.int32).reshape(2, 128)

    h0 = _tc_call(_tc1_body, (N, 64), x, w1)
    acc1, dis16 = _conv1_call(h0, row3, col3, ew3, zero64, arange2)
    dis = dis16.reshape(N, 1)
    hs2 = _tc_call(_tc2_body, (N, 32), dis, acc1, h0, w2, b1f)
    acc2 = _conv_call(32, hs2, row3, col3, ew3, zero32)
    hs3 = _tc_call(_tc3_body, (N, 16), dis, acc2, hs2, b2f, x, wd)
    acc3 = _conv_call(16, hs3, row3, col3, ew3, zero16)
    out = _tc_call(_tc4_body, (1, 1), dis, acc3, hs3, bdp, wf, bff)
    return out.reshape(()).astype(jnp.float64)
